# baseline (speedup 1.0000x reference)
import functools

import jax
import jax.numpy as jnp
from jax import lax
from jax.experimental import pallas as pl
from jax.experimental.pallas import tpu as pltpu
from jax.experimental.pallas import tpu_sc as plsc

VOCAB = 1000000
EMBED = 64
BATCH = 4096
SEQ = 200

_info = plsc.get_sparse_core_info()
NC, NS, NL = _info.num_cores, _info.num_subcores, _info.num_lanes
NW = NC * NS
COLS_PER_W = BATCH // NW
NBUF = 4
N_CHUNKS = SEQ // NBUF


@functools.partial(
    pl.kernel,
    out_type=jax.ShapeDtypeStruct((SEQ, EMBED, BATCH), jnp.float32),
    mesh=plsc.VectorSubcoreMesh(core_axis_name="c", subcore_axis_name="s"),
    scratch_types=[
        pltpu.VMEM((SEQ, COLS_PER_W), jnp.int32),
        pltpu.VMEM((NBUF, COLS_PER_W, EMBED), jnp.float32),
        pltpu.VMEM((NBUF, EMBED, COLS_PER_W), jnp.float32),
        pltpu.SemaphoreType.DMA((NBUF,)),
        pltpu.SemaphoreType.DMA((NBUF,)),
    ],
    compiler_params=pltpu.CompilerParams(
        use_tc_tiling_on_sc=False, needs_layout_passes=False
    ),
)
def _gather_kernel(idx_hbm, table_hbm, out_hbm, idx_v, gbufs, tbufs, gsems, osems):
    wid = lax.axis_index("s") * NC + lax.axis_index("c")
    base = wid * COLS_PER_W
    pltpu.sync_copy(idx_hbm.at[:, pl.ds(base, COLS_PER_W)], idx_v)

    iota = lax.iota(jnp.int32, NL)
    row_ids = [iota + c * NL for c in range(COLS_PER_W // NL)]

    def gather(t, b):
        return pltpu.make_async_copy(
            table_hbm.at[idx_v.at[t]],
            gbufs.at[b],
            gsems.at[b],
        )

    def outcopy(t, b):
        return pltpu.make_async_copy(
            tbufs.at[b],
            out_hbm.at[t, :, pl.ds(base, COLS_PER_W)],
            osems.at[b],
        )

    def transpose(b):
        def per_e(e):
            cols = jnp.full((NL,), 0, jnp.int32) + e
            for c, rows in enumerate(row_ids):
                vals = plsc.load_gather(gbufs.at[b], [rows, cols])
                tbufs[b, e, pl.ds(c * NL, NL)] = vals

        pl.loop(0, EMBED)(per_e)

    def chunk(ck):
        for b in range(NBUF):
            t = ck * NBUF + b

            @pl.when(ck > 0)
            def _():
                outcopy(t - NBUF, b).wait()

            gather(t, b).start()
        for b in range(NBUF):
            t = ck * NBUF + b
            gather(t, b).wait()
            transpose(b)
            outcopy(t, b).start()

    pl.loop(0, N_CHUNKS)(chunk)
    for b in range(NBUF):
        outcopy((N_CHUNKS - 1) * NBUF + b, b).wait()


def kernel(indices, wte):
    out_t = _gather_kernel(indices.T, wte)
    return jnp.transpose(out_t, (2, 0, 1))


# speedup vs baseline: 1.7085x; 1.7085x over previous
import functools

import jax
import jax.numpy as jnp
from jax import lax
from jax.experimental import pallas as pl
from jax.experimental.pallas import tpu as pltpu
from jax.experimental.pallas import tpu_sc as plsc

VOCAB = 1000000
EMBED = 64
BATCH = 4096
SEQ = 200

_info = plsc.get_sparse_core_info()
NC, NS = _info.num_cores, _info.num_subcores
NW = NC * NS
ROWS_PER_W = BATCH // NW
NBUF = 4
N_CHUNKS = ROWS_PER_W // NBUF


@functools.partial(
    pl.kernel,
    out_type=jax.ShapeDtypeStruct((BATCH, SEQ, EMBED), jnp.float32),
    mesh=plsc.VectorSubcoreMesh(core_axis_name="c", subcore_axis_name="s"),
    scratch_types=[
        pltpu.VMEM((ROWS_PER_W, SEQ), jnp.int32),
        pltpu.VMEM((NBUF, SEQ, EMBED), jnp.float32),
        pltpu.SemaphoreType.DMA((NBUF,)),
        pltpu.SemaphoreType.DMA((NBUF,)),
    ],
    compiler_params=pltpu.CompilerParams(use_tc_tiling_on_sc=False),
)
def _gather_kernel(idx_hbm, table_hbm, out_hbm, idx_v, bufs, gsems, osems):
    wid = lax.axis_index("s") * NC + lax.axis_index("c")
    base = wid * ROWS_PER_W
    pltpu.sync_copy(idx_hbm.at[pl.ds(base, ROWS_PER_W)], idx_v)

    def gather(r, b):
        return pltpu.make_async_copy(
            table_hbm.at[idx_v.at[r]],
            bufs.at[b],
            gsems.at[b],
        )

    def outcopy(r, b):
        return pltpu.make_async_copy(
            bufs.at[b],
            out_hbm.at[base + r],
            osems.at[b],
        )

    def chunk(c):
        for b in range(NBUF):
            r = c * NBUF + b

            @pl.when(c > 0)
            def _():
                outcopy(r - NBUF, b).wait()

            gather(r, b).start()
        for b in range(NBUF):
            r = c * NBUF + b
            gather(r, b).wait()
            outcopy(r, b).start()

    pl.loop(0, N_CHUNKS)(chunk)
    for b in range(NBUF):
        outcopy((N_CHUNKS - 1) * NBUF + b, b).wait()


def kernel(indices, wte):
    idx = jnp.minimum(jnp.maximum(indices, 0), VOCAB - 1)
    out = _gather_kernel(idx, wte)
    one = (idx[0, 0] * 0 + 1).astype(jnp.float32)
    return out * one


# speedup vs baseline: 1.7086x; 1.0001x over previous
import functools

import jax
import jax.numpy as jnp
from jax import lax
from jax.experimental import pallas as pl
from jax.experimental.pallas import tpu as pltpu
from jax.experimental.pallas import tpu_sc as plsc

VOCAB = 1000000
EMBED = 64
BATCH = 4096
SEQ = 200

_info = plsc.get_sparse_core_info()
NC, NS = _info.num_cores, _info.num_subcores
NW = NC * NS
ROWS_PER_W = BATCH // NW
NBUF = 4
N_CHUNKS = ROWS_PER_W // NBUF


@functools.partial(
    pl.kernel,
    out_type=jax.ShapeDtypeStruct((BATCH, SEQ, EMBED), jnp.float32),
    mesh=plsc.VectorSubcoreMesh(core_axis_name="c", subcore_axis_name="s"),
    scratch_types=[
        pltpu.VMEM((ROWS_PER_W * SEQ,), jnp.int32),
        pltpu.VMEM((NBUF, SEQ, EMBED), jnp.float32),
        pltpu.SemaphoreType.DMA((NBUF,)),
        pltpu.SemaphoreType.DMA((NBUF,)),
    ],
    compiler_params=pltpu.CompilerParams(use_tc_tiling_on_sc=False),
)
def _gather_kernel(idx_hbm, table_hbm, out_hbm, idx_v, bufs, gsems, osems):
    wid = lax.axis_index("s") * NC + lax.axis_index("c")
    base = wid * ROWS_PER_W
    pltpu.sync_copy(idx_hbm.at[pl.ds(base * SEQ, ROWS_PER_W * SEQ)], idx_v)

    def gather(r, b):
        return pltpu.make_async_copy(
            table_hbm.at[idx_v.at[pl.ds(r * SEQ, SEQ)]],
            bufs.at[b],
            gsems.at[b],
        )

    def outcopy(r, b):
        return pltpu.make_async_copy(
            bufs.at[b],
            out_hbm.at[base + r],
            osems.at[b],
        )

    def chunk(c):
        for b in range(NBUF):
            r = c * NBUF + b

            @pl.when(c > 0)
            def _():
                outcopy(r - NBUF, b).wait()

            gather(r, b).start()
        for b in range(NBUF):
            r = c * NBUF + b
            gather(r, b).wait()
            outcopy(r, b).start()

    pl.loop(0, N_CHUNKS)(chunk)
    for b in range(NBUF):
        outcopy((N_CHUNKS - 1) * NBUF + b, b).wait()


def kernel(indices, wte):
    idx = jnp.minimum(jnp.maximum(indices, 0), VOCAB - 1).reshape(-1)
    out = _gather_kernel(idx, wte)
    one = (idx[0] * 0 + 1).astype(jnp.float32)
    return out * one


# speedup vs baseline: 1.7096x; 1.0006x over previous
import functools

import jax
import jax.numpy as jnp
from jax import lax
from jax.experimental import pallas as pl
from jax.experimental.pallas import tpu as pltpu
from jax.experimental.pallas import tpu_sc as plsc

VOCAB = 1000000
EMBED = 64
BATCH = 4096
SEQ = 200

_info = plsc.get_sparse_core_info()
NC, NS = _info.num_cores, _info.num_subcores
NW = NC * NS
ROWS_PER_W = BATCH // NW
NBUF = 4
N_CHUNKS = ROWS_PER_W // NBUF


@functools.partial(
    pl.kernel,
    out_type=jax.ShapeDtypeStruct((BATCH, SEQ, EMBED), jnp.float32),
    mesh=plsc.VectorSubcoreMesh(core_axis_name="c", subcore_axis_name="s"),
    scratch_types=[
        pltpu.VMEM((ROWS_PER_W * SEQ,), jnp.int32),
        pltpu.VMEM((NBUF, SEQ, EMBED), jnp.float32),
        pltpu.SemaphoreType.DMA((NBUF,)),
        pltpu.SemaphoreType.DMA((NBUF,)),
    ],
    compiler_params=pltpu.CompilerParams(use_tc_tiling_on_sc=False),
)
def _gather_kernel(idx_hbm, table_hbm, out_hbm, idx_v, bufs, gsems, osems):
    wid = lax.axis_index("s") * NC + lax.axis_index("c")
    base = wid * ROWS_PER_W
    pltpu.sync_copy(idx_hbm.at[pl.ds(base * SEQ, ROWS_PER_W * SEQ)], idx_v)

    def gather(r, b):
        return pltpu.make_async_copy(
            table_hbm.at[idx_v.at[pl.ds(r * SEQ, SEQ)]],
            bufs.at[b],
            gsems.at[b],
        )

    def outcopy(r, b):
        return pltpu.make_async_copy(
            bufs.at[b],
            out_hbm.at[base + r],
            osems.at[b],
        )

    def chunk(c):
        for b in range(NBUF):
            r = c * NBUF + b

            @pl.when(c > 0)
            def _():
                outcopy(r - NBUF, b).wait()

            gather(r, b).start()
        for b in range(NBUF):
            r = c * NBUF + b
            gather(r, b).wait()
            outcopy(r, b).start()

    pl.loop(0, N_CHUNKS)(chunk)
    for b in range(NBUF):
        outcopy((N_CHUNKS - 1) * NBUF + b, b).wait()


def kernel(indices, wte):
    idx = jnp.minimum(jnp.maximum(indices, 0), VOCAB - 1).reshape(-1)
    one_w = (idx[0] * 0 + 1).astype(jnp.float32)
    out = _gather_kernel(idx, wte * one_w)
    one = (idx[0] * 0 + 1).astype(jnp.float32)
    return out * one
